# baseline (speedup 1.0000x reference)
import functools

import jax
import jax.numpy as jnp
from jax import lax
from jax.experimental import pallas as pl
from jax.experimental.pallas import tpu as pltpu
from jax.experimental.pallas import tpu_sc as plsc


def _make_emb_kernel(vocab, dim, b_tot, nw, chunk):
    b_per_w = b_tot // nw
    n_chunks = b_per_w // chunk
    mesh = plsc.VectorSubcoreMesh(core_axis_name="c", subcore_axis_name="s")

    @functools.partial(
        pl.kernel,
        mesh=mesh,
        compiler_params=pltpu.CompilerParams(use_tc_tiling_on_sc=False),
        out_type=jax.ShapeDtypeStruct((b_tot, dim), jnp.float32),
        scratch_types=[
            pltpu.VMEM((chunk,), jnp.int32),
            pltpu.VMEM((chunk, dim), jnp.float32),
            pltpu.SemaphoreType.DMA,
        ],
    )
    def emb(ids_hbm, table_hbm, out_hbm, idx_v, rows_v, sem):
        wid = lax.axis_index("s") * 2 + lax.axis_index("c")
        base = wid * b_per_w

        def body(i, carry):
            off = base + i * chunk
            pltpu.sync_copy(ids_hbm.at[pl.ds(off, chunk)], idx_v)
            pltpu.async_copy(table_hbm.at[idx_v], rows_v, sem).wait()
            pltpu.sync_copy(rows_v, out_hbm.at[pl.ds(off, chunk)])
            return carry

        lax.fori_loop(0, n_chunks, body, 0)

    return emb


def kernel(input_ids, weight):
    b, l = input_ids.shape
    vocab, dim = weight.shape
    b_tot = b * l
    nw = 32
    chunk = 512
    ids_flat = input_ids.reshape(-1).astype(jnp.int32)
    emb = _make_emb_kernel(vocab, dim, b_tot, nw, chunk)
    out = emb(ids_flat, weight)
    return out.reshape(b, l, dim)
